# baseline (speedup 1.0000x reference)
import functools

import jax
import jax.numpy as jnp
from jax import lax
from jax.experimental import pallas as pl
from jax.experimental.pallas import tpu as pltpu
from jax.experimental.pallas import tpu_sc as plsc

EMBED_DIM = 64
NUM_WORKERS = 32
RTILE = 128
NBUF = 2
RETILE_BLK = 512


def _retile_body(wt_ref, out_ref):
    out_ref[:, : EMBED_DIM] = wt_ref[...].T


def _retile(wt):
    n_rows = wt.shape[1]
    grid = (n_rows + RETILE_BLK - 1) // RETILE_BLK
    return pl.pallas_call(
        _retile_body,
        grid=(grid,),
        in_specs=[pl.BlockSpec((EMBED_DIM, RETILE_BLK), lambda i: (0, i))],
        out_specs=pl.BlockSpec((RETILE_BLK, 2 * EMBED_DIM), lambda i: (i, 0)),
        out_shape=jax.ShapeDtypeStruct((n_rows, 2 * EMBED_DIM), jnp.float32),
    )(wt)


@functools.partial(jax.jit, static_argnames=("seq_len",))
def _gather(weight, idx, seq_len):
    n_ctiles = EMBED_DIM // 8
    mesh = plsc.VectorSubcoreMesh(core_axis_name="c", subcore_axis_name="s")

    @functools.partial(
        pl.kernel,
        mesh=mesh,
        out_type=jax.ShapeDtypeStruct(
            (seq_len, n_ctiles, NUM_WORKERS, 8, RTILE), jnp.float32),
        scratch_types=[
            pltpu.VMEM((seq_len, RTILE), jnp.int32),
            [pltpu.VMEM((RTILE, EMBED_DIM), jnp.float32)
             for _ in range(NBUF)],
            [pltpu.VMEM((EMBED_DIM, RTILE), jnp.float32)
             for _ in range(NBUF)],
            [pltpu.SemaphoreType.DMA for _ in range(NBUF)],
            [pltpu.SemaphoreType.DMA for _ in range(NBUF)],
        ],
        compiler_params=pltpu.CompilerParams(
            use_tc_tiling_on_sc=False, needs_layout_passes=False),
    )
    def body(weight_hbm, idx_hbm, out_hbm, idx_v, gbuf, tbuf, gsems, wsems):
        wid = lax.axis_index("s") * 2 + lax.axis_index("c")
        pltpu.sync_copy(idx_hbm.at[wid], idx_v)
        lane = lax.iota(jnp.int32, 16)

        def gather_copy(s, b):
            return pltpu.make_async_copy(
                weight_hbm.at[idx_v.at[s]], gbuf[b], gsems[b])

        def write_copies(s, b):
            return [
                pltpu.make_async_copy(
                    tbuf[b].at[pl.ds(ct * 8, 8)],
                    out_hbm.at[s, ct, wid],
                    wsems[b])
                for ct in range(n_ctiles)
            ]

        n_cblk = EMBED_DIM // 16

        def transpose(b):
            g, t = gbuf[b], tbuf[b]

            @plsc.parallel_loop(0, (RTILE // 16) * n_cblk, 1, unroll=2)
            def blk(q):
                r0 = (q // n_cblk) * 16
                c0 = lax.rem(q, n_cblk) * 16
                rows = r0 + lane
                for d in range(16):
                    cols = c0 + ((lane + d) & 15)
                    v = plsc.load_gather(g, [rows, cols])
                    plsc.store_scatter(t, [cols, rows], v)

        gather_copy(0, 0).start()

        def pair(s2, carry):
            for b in range(NBUF):
                s = s2 * NBUF + b

                @pl.when(s + 1 < seq_len)
                def _():
                    gather_copy(s + 1, (b + 1) % NBUF).start()

                gather_copy(s, b).wait()

                @pl.when(s >= NBUF)
                def _():
                    for w in write_copies(s - NBUF, b):
                        w.wait()

                transpose(b)
                for w in write_copies(s, b):
                    w.start()
            return carry

        lax.fori_loop(0, seq_len // NBUF, pair, 0)

        for b in range(NBUF):
            for w in write_copies(seq_len - NBUF + b, b):
                w.wait()

    return body(weight, idx)


def kernel(x, weight):
    n_x_rows, seq_len = x.shape
    n_table_rows = weight.shape[0]
    table = _retile(weight.T).reshape(2 * n_table_rows, EMBED_DIM)
    idx = ((x.astype(jnp.int32) * 2)
           .reshape(NUM_WORKERS, RTILE, seq_len)
           .transpose(0, 2, 1))
    out5 = _gather(table, idx, seq_len)
    return (out5.transpose(2, 4, 0, 1, 3)
            .reshape(n_x_rows, seq_len, EMBED_DIM))


# speedup vs baseline: 1.5989x; 1.5989x over previous
import functools

import jax
import jax.numpy as jnp
from jax import lax
from jax.experimental import pallas as pl
from jax.experimental.pallas import tpu as pltpu
from jax.experimental.pallas import tpu_sc as plsc

EMBED_DIM = 64
NUM_WORKERS = 32
RTILE = 128
NBUF = 2
RETILE_BLK = 2048


def _retile_body(wt_ref, out_ref):
    eye = jnp.eye(EMBED_DIM, dtype=jnp.float32)
    out_ref[:, : EMBED_DIM] = jax.lax.dot_general(
        wt_ref[...], eye, (((0,), (0,)), ((), ())),
        preferred_element_type=jnp.float32)


def _retile(wt):
    n_rows = wt.shape[1]
    grid = (n_rows + RETILE_BLK - 1) // RETILE_BLK
    return pl.pallas_call(
        _retile_body,
        grid=(grid,),
        in_specs=[pl.BlockSpec((EMBED_DIM, RETILE_BLK), lambda i: (0, i))],
        out_specs=pl.BlockSpec((RETILE_BLK, 2 * EMBED_DIM), lambda i: (i, 0)),
        out_shape=jax.ShapeDtypeStruct((n_rows, 2 * EMBED_DIM), jnp.float32),
    )(wt)


@functools.partial(jax.jit, static_argnames=("seq_len",))
def _gather(weight, idx, seq_len):
    n_ctiles = EMBED_DIM // 8
    mesh = plsc.VectorSubcoreMesh(core_axis_name="c", subcore_axis_name="s")

    @functools.partial(
        pl.kernel,
        mesh=mesh,
        out_type=jax.ShapeDtypeStruct(
            (seq_len, n_ctiles, NUM_WORKERS, 8, RTILE), jnp.float32),
        scratch_types=[
            pltpu.VMEM((seq_len, RTILE), jnp.int32),
            [pltpu.VMEM((RTILE, EMBED_DIM), jnp.float32)
             for _ in range(NBUF)],
            [pltpu.VMEM((EMBED_DIM, RTILE), jnp.float32)
             for _ in range(NBUF)],
            [pltpu.SemaphoreType.DMA for _ in range(NBUF)],
            [pltpu.SemaphoreType.DMA for _ in range(NBUF)],
        ],
        compiler_params=pltpu.CompilerParams(
            use_tc_tiling_on_sc=False, needs_layout_passes=False),
    )
    def body(weight_hbm, idx_hbm, out_hbm, idx_v, gbuf, tbuf, gsems, wsems):
        wid = lax.axis_index("s") * 2 + lax.axis_index("c")
        pltpu.sync_copy(idx_hbm.at[wid], idx_v)
        lane = lax.iota(jnp.int32, 16)

        def gather_copy(s, b):
            return pltpu.make_async_copy(
                weight_hbm.at[idx_v.at[s]], gbuf[b], gsems[b])

        def write_copies(s, b):
            return [
                pltpu.make_async_copy(
                    tbuf[b].at[pl.ds(ct * 8, 8)],
                    out_hbm.at[s, ct, wid],
                    wsems[b])
                for ct in range(n_ctiles)
            ]

        n_cblk = EMBED_DIM // 16

        def transpose(b):
            g, t = gbuf[b], tbuf[b]

            @plsc.parallel_loop(0, (RTILE // 16) * n_cblk, 1, unroll=2)
            def blk(q):
                r0 = (q // n_cblk) * 16
                c0 = lax.rem(q, n_cblk) * 16
                rows = r0 + lane
                for d in range(16):
                    cols = c0 + ((lane + d) & 15)
                    v = plsc.load_gather(g, [rows, cols])
                    plsc.store_scatter(t, [cols, rows], v)

        gather_copy(0, 0).start()

        def pair(s2, carry):
            for b in range(NBUF):
                s = s2 * NBUF + b

                @pl.when(s + 1 < seq_len)
                def _():
                    gather_copy(s + 1, (b + 1) % NBUF).start()

                gather_copy(s, b).wait()

                @pl.when(s >= NBUF)
                def _():
                    for w in write_copies(s - NBUF, b):
                        w.wait()

                transpose(b)
                for w in write_copies(s, b):
                    w.start()
            return carry

        lax.fori_loop(0, seq_len // NBUF, pair, 0)

        for b in range(NBUF):
            for w in write_copies(seq_len - NBUF + b, b):
                w.wait()

    return body(weight, idx)


def kernel(x, weight):
    n_x_rows, seq_len = x.shape
    n_table_rows = weight.shape[0]
    table = _retile(weight.T).reshape(2 * n_table_rows, EMBED_DIM)
    idx = ((x.astype(jnp.int32) * 2)
           .reshape(NUM_WORKERS, RTILE, seq_len)
           .transpose(0, 2, 1))
    out5 = _gather(table, idx, seq_len)
    return (out5.transpose(2, 4, 0, 1, 3)
            .reshape(n_x_rows, seq_len, EMBED_DIM))
